# baseline (speedup 1.0000x reference)
import functools

import jax
import jax.numpy as jnp
from jax import lax
from jax.experimental import pallas as pl
from jax.experimental.pallas import tpu as pltpu
from jax.experimental.pallas import tpu_sc as plsc

N = 10000
NP = 10240
E = 320000
F_IN = 128
H = 64
C = 40
FP = 64

NC = 2
NS = 16
NW = NC * NS
EPT = E // NW
K = 200
NCHUNK = EPT // K
RPT = NP // NS

_mesh = plsc.VectorSubcoreMesh(core_axis_name="c", subcore_axis_name="s")


@functools.partial(
    pl.kernel,
    out_type=jax.ShapeDtypeStruct((NC, NP, 16), jnp.float32),
    mesh=_mesh,
    compiler_params=pltpu.CompilerParams(use_tc_tiling_on_sc=False),
    scratch_types=[
        pltpu.VMEM((EPT,), jnp.int32),
        pltpu.VMEM((K, 16), jnp.float32),
        pltpu.VMEM_SHARED((NP, 16), jnp.float32),
        pltpu.SemaphoreType.DMA,
    ],
)
def _sc_degree(ei_hbm, ones_hbm, z16_hbm, out_hbm, colv, onesv, acc, sem):
    c = lax.axis_index("c")
    s = lax.axis_index("s")
    wid = c * NS + s
    pltpu.sync_copy(z16_hbm.at[pl.ds(s * RPT, RPT)], acc.at[pl.ds(s * RPT, RPT)])
    pltpu.sync_copy(ones_hbm, onesv)
    pltpu.sync_copy(ei_hbm.at[pl.ds(E + wid * EPT, EPT)], colv)
    plsc.subcore_barrier()

    @pl.loop(0, NCHUNK, step=5)
    def _group(g):
        for b in range(5):
            pltpu.async_copy(onesv, acc.at[colv.at[pl.ds((g + b) * K, K)]],
                             sem, add=True)
        for b in range(5):
            pltpu.make_async_copy(onesv, acc.at[colv.at[pl.ds((g + b) * K, K)]],
                                  sem).wait()

    plsc.subcore_barrier()
    pltpu.sync_copy(acc.at[pl.ds(s * RPT, RPT)], out_hbm.at[c, pl.ds(s * RPT, RPT)])


@functools.partial(
    pl.kernel,
    out_type=jax.ShapeDtypeStruct((NC, NP, FP), jnp.float32),
    mesh=_mesh,
    compiler_params=pltpu.CompilerParams(use_tc_tiling_on_sc=False),
    scratch_types=[
        pltpu.VMEM((EPT,), jnp.int32),
        pltpu.VMEM((EPT,), jnp.int32),
        pltpu.VMEM((K, FP), jnp.float32),
        pltpu.VMEM((K, FP), jnp.float32),
        pltpu.VMEM_SHARED((NP, FP), jnp.float32),
        pltpu.SemaphoreType.DMA,
        pltpu.SemaphoreType.DMA,
        pltpu.SemaphoreType.DMA,
        pltpu.SemaphoreType.DMA,
    ],
)
def _sc_msg(hp_hbm, ei_hbm, z64_hbm, out_hbm,
            rowv, colv, buf_a, buf_b, acc, sem_a, sem_b, sem_sa, sem_sb):
    c = lax.axis_index("c")
    s = lax.axis_index("s")
    wid = c * NS + s
    pltpu.sync_copy(z64_hbm.at[pl.ds(s * RPT, RPT)], acc.at[pl.ds(s * RPT, RPT)])
    pltpu.sync_copy(ei_hbm.at[pl.ds(wid * EPT, EPT)], rowv)
    pltpu.sync_copy(ei_hbm.at[pl.ds(E + wid * EPT, EPT)], colv)
    plsc.subcore_barrier()

    def _gather(j, buf, sem):
        pltpu.async_copy(hp_hbm.at[rowv.at[pl.ds(j * K, K)]], buf, sem)

    def _gather_wait(j, buf, sem):
        pltpu.make_async_copy(hp_hbm.at[rowv.at[pl.ds(j * K, K)]], buf, sem).wait()

    def _scatter(j, buf, sem):
        pltpu.async_copy(buf, acc.at[colv.at[pl.ds(j * K, K)]], sem, add=True)

    def _scatter_wait(j, buf, sem):
        pltpu.make_async_copy(buf, acc.at[colv.at[pl.ds(j * K, K)]], sem).wait()

    _gather(0, buf_a, sem_a)
    _gather(1, buf_b, sem_b)

    NPAIR = NCHUNK - (NCHUNK % 2)
    @pl.loop(0, NPAIR, step=2)
    def _pair(j):
        _gather_wait(j, buf_a, sem_a)
        _scatter(j, buf_a, sem_sa)
        _gather_wait(j + 1, buf_b, sem_b)
        _scatter(j + 1, buf_b, sem_sb)
        _scatter_wait(j, buf_a, sem_sa)

        @pl.when(j + 2 < NCHUNK)
        def _():
            _gather(j + 2, buf_a, sem_a)

        _scatter_wait(j + 1, buf_b, sem_sb)

        @pl.when(j + 3 < NCHUNK)
        def _():
            _gather(j + 3, buf_b, sem_b)

    if NCHUNK % 2:
        _gather_wait(NCHUNK - 1, buf_a, sem_a)
        pltpu.sync_copy(buf_a, acc.at[colv.at[pl.ds((NCHUNK - 1) * K, K)]],
                        add=True)

    plsc.subcore_barrier()
    pltpu.sync_copy(acc.at[pl.ds(s * RPT, RPT)], out_hbm.at[c, pl.ds(s * RPT, RPT)])


NP2 = NP // 2
_BLK = 1024
_GRID = NP2 // _BLK


def _prep_body(deg_ref, x2_ref, w1d_ref, rex_ref, dis_ref, hp_ref):
    d32 = 1.0 + deg_ref[0] + deg_ref[1]
    dis32 = lax.rsqrt(d32)
    dis = jnp.dot(dis32, rex_ref[...], preferred_element_type=jnp.float32)
    dis_ref[...] = dis
    h = jnp.dot(x2_ref[...], w1d_ref[...], preferred_element_type=jnp.float32,
                precision=lax.Precision.HIGHEST)
    hp_ref[...] = dis * h


def _combine_body(acc_ref, hp_ref, dis_ref, b_ref, wd_ref, hpn_ref):
    dis = dis_ref[...]
    h = dis * (acc_ref[0] + acc_ref[1] + hp_ref[...]) + b_ref[...]
    h = jnp.maximum(h, 0.0)
    hn = jnp.dot(h, wd_ref[...], preferred_element_type=jnp.float32,
                 precision=lax.Precision.HIGHEST)
    hpn_ref[...] = dis * hn


def _final_body(acc_ref, hp_ref, dis_ref, b_ref, out_ref):
    logits = dis_ref[...] * (acc_ref[0] + acc_ref[1] + hp_ref[...]) + b_ref[...]
    lanes = lax.broadcasted_iota(jnp.int32, logits.shape, 1)
    left = lanes < FP
    valid = (lanes % FP) < C
    neg = jnp.float32(-jnp.inf)
    lm = jnp.where(valid, logits, neg)
    m_l = jnp.max(jnp.where(left, lm, neg), axis=1, keepdims=True)
    m_r = jnp.max(jnp.where(left, neg, lm), axis=1, keepdims=True)
    m = jnp.where(left, m_l, m_r)
    ex = jnp.where(valid, jnp.exp(logits - m), 0.0)
    s_l = jnp.sum(jnp.where(left, ex, 0.0), axis=1, keepdims=True)
    s_r = jnp.sum(jnp.where(left, 0.0, ex), axis=1, keepdims=True)
    lse = jnp.log(jnp.where(left, s_l, s_r)) + m
    out_ref[...] = logits - lse


def _row_spec(f):
    return pl.BlockSpec((_BLK, f), lambda i: (i, 0))


def _full_spec(shape):
    return pl.BlockSpec(shape, lambda i: tuple(0 for _ in shape))


def _tc_prep(deg2, x2, w1d, rex):
    return pl.pallas_call(
        _prep_body,
        grid=(_GRID,),
        in_specs=[pl.BlockSpec((2, _BLK, 32), lambda i: (0, i, 0)),
                  _row_spec(2 * F_IN), _full_spec((2 * F_IN, 128)),
                  _full_spec((32, 128))],
        out_specs=[_row_spec(128), _row_spec(128)],
        out_shape=[jax.ShapeDtypeStruct((NP2, 128), jnp.float32),
                   jax.ShapeDtypeStruct((NP2, 128), jnp.float32)],
    )(deg2, x2, w1d, rex)


def _tc_combine(acc2, hp, dis, b, wd):
    return pl.pallas_call(
        _combine_body,
        grid=(_GRID,),
        in_specs=[pl.BlockSpec((2, _BLK, 128), lambda i: (0, i, 0)),
                  _row_spec(128), _row_spec(128), _full_spec((1, 128)),
                  _full_spec((128, 128))],
        out_specs=[_row_spec(128)],
        out_shape=[jax.ShapeDtypeStruct((NP2, 128), jnp.float32)],
    )(acc2, hp, dis, b, wd)[0]


def _tc_final(acc2, hp, dis, b):
    return pl.pallas_call(
        _final_body,
        grid=(_GRID,),
        in_specs=[pl.BlockSpec((2, _BLK, 128), lambda i: (0, i, 0)),
                  _row_spec(128), _row_spec(128), _full_spec((1, 128))],
        out_specs=[_row_spec(128)],
        out_shape=[jax.ShapeDtypeStruct((NP2, 128), jnp.float32)],
    )(acc2, hp, dis, b)[0]


def _blockdiag(w):
    fi, fo = w.shape
    z = jnp.zeros((fi, fo), jnp.float32)
    return jnp.concatenate([
        jnp.concatenate([w, z], axis=1),
        jnp.concatenate([z, w], axis=1),
    ], axis=0)


def kernel(x, edge_index, W1, b1, W2, b2, W3, b3):
    ones16 = jnp.ones((K, 16), jnp.float32)
    z16 = jnp.zeros((NP, 16), jnp.float32)
    z64 = jnp.zeros((NP, FP), jnp.float32)
    x2 = jnp.pad(x, ((0, NP - N), (0, 0))).reshape(NP2, 2 * F_IN)
    w3p = jnp.pad(W3, ((0, 0), (0, FP - C)))
    w1d = _blockdiag(W1)
    w2d = _blockdiag(W2)
    w3d = _blockdiag(w3p)
    rex = jnp.zeros((32, 128), jnp.float32)
    rex = rex.at[0, :64].set(1.0).at[16, 64:].set(1.0)
    b1p = jnp.concatenate([b1, b1]).reshape(1, 128)
    b2p = jnp.concatenate([b2, b2]).reshape(1, 128)
    b3f = jnp.pad(b3, (0, FP - C))
    b3p = jnp.concatenate([b3f, b3f]).reshape(1, 128)

    eif = edge_index.reshape(2 * E)
    deg = _sc_degree(eif, ones16, z16)
    deg2 = deg.reshape(NC, NP2, 32)
    dis, hp1 = _tc_prep(deg2, x2, w1d, rex)

    acc1 = _sc_msg(hp1.reshape(NP, FP), eif, z64)
    hp2 = _tc_combine(acc1.reshape(NC, NP2, 128), hp1, dis, b1p, w2d)

    acc2 = _sc_msg(hp2.reshape(NP, FP), eif, z64)
    hp3 = _tc_combine(acc2.reshape(NC, NP2, 128), hp2, dis, b2p, w3d)

    acc3 = _sc_msg(hp3.reshape(NP, FP), eif, z64)
    out = _tc_final(acc3.reshape(NC, NP2, 128), hp3, dis, b3p)
    return out.reshape(NP, FP)[:N, :C]


# speedup vs baseline: 1.2140x; 1.2140x over previous
import functools

import jax
import jax.numpy as jnp
from jax import lax
from jax.experimental import pallas as pl
from jax.experimental.pallas import tpu as pltpu
from jax.experimental.pallas import tpu_sc as plsc

N = 10000
NP = 10240
E = 320000
F_IN = 128
H = 64
C = 40
FP = 64

NC = 2
NS = 16
NW = NC * NS
EPT = E // NW
K = 200
NCHUNK = EPT // K
RPT = NP // NS

_mesh = plsc.VectorSubcoreMesh(core_axis_name="c", subcore_axis_name="s")


@functools.partial(
    pl.kernel,
    out_type=jax.ShapeDtypeStruct((NC, NP, 16), jnp.float32),
    mesh=_mesh,
    compiler_params=pltpu.CompilerParams(use_tc_tiling_on_sc=False),
    scratch_types=[
        pltpu.VMEM((EPT,), jnp.int32),
        pltpu.VMEM((K, 16), jnp.float32),
        pltpu.VMEM_SHARED((NP, 16), jnp.float32),
        pltpu.SemaphoreType.DMA,
    ],
)
def _sc_degree(ei_hbm, ones_hbm, z16_hbm, out_hbm, colv, onesv, acc, sem):
    c = lax.axis_index("c")
    s = lax.axis_index("s")
    wid = c * NS + s
    pltpu.sync_copy(z16_hbm.at[pl.ds(s * RPT, RPT)], acc.at[pl.ds(s * RPT, RPT)])
    pltpu.sync_copy(ones_hbm, onesv)
    pltpu.sync_copy(ei_hbm.at[pl.ds(E + wid * EPT, EPT)], colv)
    plsc.subcore_barrier()

    @pl.loop(0, NCHUNK, step=5)
    def _group(g):
        for b in range(5):
            pltpu.async_copy(onesv, acc.at[colv.at[pl.ds((g + b) * K, K)]],
                             sem, add=True)
        for b in range(5):
            pltpu.make_async_copy(onesv, acc.at[colv.at[pl.ds((g + b) * K, K)]],
                                  sem).wait()

    plsc.subcore_barrier()
    pltpu.sync_copy(acc.at[pl.ds(s * RPT, RPT)], out_hbm.at[c, pl.ds(s * RPT, RPT)])


@functools.partial(
    pl.kernel,
    out_type=jax.ShapeDtypeStruct((NC, NP, FP), jnp.float32),
    mesh=_mesh,
    compiler_params=pltpu.CompilerParams(use_tc_tiling_on_sc=False),
    scratch_types=[
        pltpu.VMEM((EPT,), jnp.int32),
        pltpu.VMEM((EPT,), jnp.int32),
        pltpu.VMEM((K, FP), jnp.float32),
        pltpu.VMEM((K, FP), jnp.float32),
        pltpu.VMEM_SHARED((NP, FP), jnp.float32),
        pltpu.SemaphoreType.DMA,
        pltpu.SemaphoreType.DMA,
    ],
)
def _sc_msg(hp_hbm, ei_hbm, z64_hbm, out_hbm,
            rowv, colv, buf_a, buf_b, acc, sem_a, sem_b):
    c = lax.axis_index("c")
    s = lax.axis_index("s")
    wid = c * NS + s
    pltpu.sync_copy(z64_hbm.at[pl.ds(s * RPT, RPT)], acc.at[pl.ds(s * RPT, RPT)])
    pltpu.sync_copy(ei_hbm.at[pl.ds(wid * EPT, EPT)], rowv)
    pltpu.sync_copy(ei_hbm.at[pl.ds(E + wid * EPT, EPT)], colv)
    plsc.subcore_barrier()

    def _gather(j, buf, sem):
        pltpu.async_copy(hp_hbm.at[rowv.at[pl.ds(j * K, K)]], buf, sem)

    def _gather_wait(j, buf, sem):
        pltpu.make_async_copy(hp_hbm.at[rowv.at[pl.ds(j * K, K)]], buf, sem).wait()

    def _scatter_sync(j, buf):
        pltpu.sync_copy(buf, acc.at[colv.at[pl.ds(j * K, K)]], add=True)

    _gather(0, buf_a, sem_a)

    @pl.loop(0, NCHUNK, step=2)
    def _pair(j):
        _gather(j + 1, buf_b, sem_b)
        _gather_wait(j, buf_a, sem_a)
        _scatter_sync(j, buf_a)

        @pl.when(j + 2 < NCHUNK)
        def _():
            _gather(j + 2, buf_a, sem_a)

        _gather_wait(j + 1, buf_b, sem_b)
        _scatter_sync(j + 1, buf_b)

    plsc.subcore_barrier()
    pltpu.sync_copy(acc.at[pl.ds(s * RPT, RPT)], out_hbm.at[c, pl.ds(s * RPT, RPT)])


NP2 = NP // 2
_BLK = 1024
_GRID = NP2 // _BLK


def _prep_body(deg_ref, x2_ref, w1d_ref, rex_ref, dis_ref, hp_ref):
    d32 = 1.0 + deg_ref[0] + deg_ref[1]
    dis32 = lax.rsqrt(d32)
    dis = jnp.dot(dis32, rex_ref[...], preferred_element_type=jnp.float32)
    dis_ref[...] = dis
    h = jnp.dot(x2_ref[...], w1d_ref[...], preferred_element_type=jnp.float32,
                precision=lax.Precision.HIGHEST)
    hp_ref[...] = dis * h


def _combine_body(acc_ref, hp_ref, dis_ref, b_ref, wd_ref, hpn_ref):
    dis = dis_ref[...]
    h = dis * (acc_ref[0] + acc_ref[1] + hp_ref[...]) + b_ref[...]
    h = jnp.maximum(h, 0.0)
    hn = jnp.dot(h, wd_ref[...], preferred_element_type=jnp.float32,
                 precision=lax.Precision.HIGHEST)
    hpn_ref[...] = dis * hn


def _final_body(acc_ref, hp_ref, dis_ref, b_ref, out_ref):
    logits = dis_ref[...] * (acc_ref[0] + acc_ref[1] + hp_ref[...]) + b_ref[...]
    lanes = lax.broadcasted_iota(jnp.int32, logits.shape, 1)
    left = lanes < FP
    valid = (lanes % FP) < C
    neg = jnp.float32(-jnp.inf)
    lm = jnp.where(valid, logits, neg)
    m_l = jnp.max(jnp.where(left, lm, neg), axis=1, keepdims=True)
    m_r = jnp.max(jnp.where(left, neg, lm), axis=1, keepdims=True)
    m = jnp.where(left, m_l, m_r)
    ex = jnp.where(valid, jnp.exp(logits - m), 0.0)
    s_l = jnp.sum(jnp.where(left, ex, 0.0), axis=1, keepdims=True)
    s_r = jnp.sum(jnp.where(left, 0.0, ex), axis=1, keepdims=True)
    lse = jnp.log(jnp.where(left, s_l, s_r)) + m
    out_ref[...] = logits - lse


def _row_spec(f):
    return pl.BlockSpec((_BLK, f), lambda i: (i, 0))


def _full_spec(shape):
    return pl.BlockSpec(shape, lambda i: tuple(0 for _ in shape))


def _tc_prep(deg2, x2, w1d, rex):
    return pl.pallas_call(
        _prep_body,
        grid=(_GRID,),
        in_specs=[pl.BlockSpec((2, _BLK, 32), lambda i: (0, i, 0)),
                  _row_spec(2 * F_IN), _full_spec((2 * F_IN, 128)),
                  _full_spec((32, 128))],
        out_specs=[_row_spec(128), _row_spec(128)],
        out_shape=[jax.ShapeDtypeStruct((NP2, 128), jnp.float32),
                   jax.ShapeDtypeStruct((NP2, 128), jnp.float32)],
    )(deg2, x2, w1d, rex)


def _tc_combine(acc2, hp, dis, b, wd):
    return pl.pallas_call(
        _combine_body,
        grid=(_GRID,),
        in_specs=[pl.BlockSpec((2, _BLK, 128), lambda i: (0, i, 0)),
                  _row_spec(128), _row_spec(128), _full_spec((1, 128)),
                  _full_spec((128, 128))],
        out_specs=[_row_spec(128)],
        out_shape=[jax.ShapeDtypeStruct((NP2, 128), jnp.float32)],
    )(acc2, hp, dis, b, wd)[0]


def _tc_final(acc2, hp, dis, b):
    return pl.pallas_call(
        _final_body,
        grid=(_GRID,),
        in_specs=[pl.BlockSpec((2, _BLK, 128), lambda i: (0, i, 0)),
                  _row_spec(128), _row_spec(128), _full_spec((1, 128))],
        out_specs=[_row_spec(128)],
        out_shape=[jax.ShapeDtypeStruct((NP2, 128), jnp.float32)],
    )(acc2, hp, dis, b)[0]


def _blockdiag(w):
    fi, fo = w.shape
    z = jnp.zeros((fi, fo), jnp.float32)
    return jnp.concatenate([
        jnp.concatenate([w, z], axis=1),
        jnp.concatenate([z, w], axis=1),
    ], axis=0)


def kernel(x, edge_index, W1, b1, W2, b2, W3, b3):
    ones16 = jnp.ones((K, 16), jnp.float32)
    z16 = jnp.zeros((NP, 16), jnp.float32)
    z64 = jnp.zeros((NP, FP), jnp.float32)
    x2 = jnp.pad(x, ((0, NP - N), (0, 0))).reshape(NP2, 2 * F_IN)
    w3p = jnp.pad(W3, ((0, 0), (0, FP - C)))
    w1d = _blockdiag(W1)
    w2d = _blockdiag(W2)
    w3d = _blockdiag(w3p)
    rex = jnp.zeros((32, 128), jnp.float32)
    rex = rex.at[0, :64].set(1.0).at[16, 64:].set(1.0)
    b1p = jnp.concatenate([b1, b1]).reshape(1, 128)
    b2p = jnp.concatenate([b2, b2]).reshape(1, 128)
    b3f = jnp.pad(b3, (0, FP - C))
    b3p = jnp.concatenate([b3f, b3f]).reshape(1, 128)

    eif = edge_index.reshape(2 * E)
    deg = _sc_degree(eif, ones16, z16)
    deg2 = deg.reshape(NC, NP2, 32)
    dis, hp1 = _tc_prep(deg2, x2, w1d, rex)

    acc1 = _sc_msg(hp1.reshape(NP, FP), eif, z64)
    hp2 = _tc_combine(acc1.reshape(NC, NP2, 128), hp1, dis, b1p, w2d)

    acc2 = _sc_msg(hp2.reshape(NP, FP), eif, z64)
    hp3 = _tc_combine(acc2.reshape(NC, NP2, 128), hp2, dis, b2p, w3d)

    acc3 = _sc_msg(hp3.reshape(NP, FP), eif, z64)
    out = _tc_final(acc3.reshape(NC, NP2, 128), hp3, dis, b3p)
    return out.reshape(NP, FP)[:N, :C]


# speedup vs baseline: 1.2458x; 1.0262x over previous
import functools

import jax
import jax.numpy as jnp
from jax import lax
from jax.experimental import pallas as pl
from jax.experimental.pallas import tpu as pltpu
from jax.experimental.pallas import tpu_sc as plsc

N = 10000
NP = 10240
E = 320000
F_IN = 128
H = 64
C = 40
FP = 64

NC = 2
NS = 16
NW = NC * NS
EPT = E // NW
K = 200
NCHUNK = EPT // K
RPT = NP // NS

_mesh = plsc.VectorSubcoreMesh(core_axis_name="c", subcore_axis_name="s")


@functools.partial(
    pl.kernel,
    out_type=jax.ShapeDtypeStruct((NC, NP, 16), jnp.float32),
    mesh=_mesh,
    compiler_params=pltpu.CompilerParams(use_tc_tiling_on_sc=False),
    scratch_types=[
        pltpu.VMEM((EPT,), jnp.int32),
        pltpu.VMEM((K, 16), jnp.float32),
        pltpu.VMEM_SHARED((NP, 16), jnp.float32),
        pltpu.SemaphoreType.DMA,
    ],
)
def _sc_degree(ei_hbm, ones_hbm, z16_hbm, out_hbm, colv, onesv, acc, sem):
    c = lax.axis_index("c")
    s = lax.axis_index("s")
    wid = c * NS + s
    pltpu.sync_copy(z16_hbm.at[pl.ds(s * RPT, RPT)], acc.at[pl.ds(s * RPT, RPT)])
    pltpu.sync_copy(ones_hbm, onesv)
    pltpu.sync_copy(ei_hbm.at[pl.ds(E + wid * EPT, EPT)], colv)
    plsc.subcore_barrier()

    @pl.loop(0, NCHUNK, step=5)
    def _group(g):
        for b in range(5):
            pltpu.async_copy(onesv, acc.at[colv.at[pl.ds((g + b) * K, K)]],
                             sem, add=True)
        for b in range(5):
            pltpu.make_async_copy(onesv, acc.at[colv.at[pl.ds((g + b) * K, K)]],
                                  sem).wait()

    plsc.subcore_barrier()
    pltpu.sync_copy(acc.at[pl.ds(s * RPT, RPT)], out_hbm.at[c, pl.ds(s * RPT, RPT)])


@functools.partial(
    pl.kernel,
    out_type=jax.ShapeDtypeStruct((NC, NP, FP), jnp.float32),
    mesh=_mesh,
    compiler_params=pltpu.CompilerParams(use_tc_tiling_on_sc=False),
    scratch_types=[
        pltpu.VMEM((EPT,), jnp.int32),
        pltpu.VMEM((EPT,), jnp.int32),
        pltpu.VMEM((K, FP), jnp.float32),
        pltpu.VMEM((K, FP), jnp.float32),
        pltpu.VMEM_SHARED((NP, FP), jnp.float32),
        pltpu.SemaphoreType.DMA,
        pltpu.SemaphoreType.DMA,
    ],
)
def _sc_msg(hp_hbm, ei_hbm, z64_hbm, out_hbm,
            rowv, colv, buf_a, buf_b, acc, sem_a, sem_b):
    c = lax.axis_index("c")
    s = lax.axis_index("s")
    wid = c * NS + s
    pltpu.sync_copy(z64_hbm.at[pl.ds(s * RPT, RPT)], acc.at[pl.ds(s * RPT, RPT)])
    pltpu.sync_copy(ei_hbm.at[pl.ds(wid * EPT, EPT)], rowv)
    pltpu.sync_copy(ei_hbm.at[pl.ds(E + wid * EPT, EPT)], colv)
    plsc.subcore_barrier()

    def _gather(j, buf, sem):
        pltpu.async_copy(hp_hbm.at[rowv.at[pl.ds(j * K, K)]], buf, sem)

    def _gather_wait(j, buf, sem):
        pltpu.make_async_copy(hp_hbm.at[rowv.at[pl.ds(j * K, K)]], buf, sem).wait()

    def _scatter_sync(j, buf):
        pltpu.sync_copy(buf, acc.at[colv.at[pl.ds(j * K, K)]], add=True)

    _gather(0, buf_a, sem_a)

    @pl.loop(0, NCHUNK, step=2)
    def _pair(j):
        _gather(j + 1, buf_b, sem_b)
        _gather_wait(j, buf_a, sem_a)
        _scatter_sync(j, buf_a)

        @pl.when(j + 2 < NCHUNK)
        def _():
            _gather(j + 2, buf_a, sem_a)

        _gather_wait(j + 1, buf_b, sem_b)
        _scatter_sync(j + 1, buf_b)

    plsc.subcore_barrier()
    pltpu.sync_copy(acc.at[pl.ds(s * RPT, RPT)], out_hbm.at[c, pl.ds(s * RPT, RPT)])


NP2 = NP // 2
_BLK = 2560
_GRID = NP2 // _BLK


def _prep_body(deg_ref, x2_ref, w1d_ref, rex_ref, dis_ref, hp_ref):
    d32 = 1.0 + deg_ref[0] + deg_ref[1]
    dis32 = lax.rsqrt(d32)
    dis = jnp.dot(dis32, rex_ref[...], preferred_element_type=jnp.float32)
    dis_ref[...] = dis
    h = jnp.dot(x2_ref[...], w1d_ref[...], preferred_element_type=jnp.float32,
                precision=lax.Precision.HIGHEST)
    hp_ref[...] = dis * h


def _combine_body(acc_ref, hp_ref, dis_ref, b_ref, wd_ref, hpn_ref):
    dis = dis_ref[...]
    h = dis * (acc_ref[0] + acc_ref[1] + hp_ref[...]) + b_ref[...]
    h = jnp.maximum(h, 0.0)
    hn = jnp.dot(h, wd_ref[...], preferred_element_type=jnp.float32,
                 precision=lax.Precision.HIGHEST)
    hpn_ref[...] = dis * hn


def _final_body(acc_ref, hp_ref, dis_ref, b_ref, out_ref):
    logits = dis_ref[...] * (acc_ref[0] + acc_ref[1] + hp_ref[...]) + b_ref[...]
    lanes = lax.broadcasted_iota(jnp.int32, logits.shape, 1)
    left = lanes < FP
    valid = (lanes % FP) < C
    neg = jnp.float32(-jnp.inf)
    lm = jnp.where(valid, logits, neg)
    m_l = jnp.max(jnp.where(left, lm, neg), axis=1, keepdims=True)
    m_r = jnp.max(jnp.where(left, neg, lm), axis=1, keepdims=True)
    m = jnp.where(left, m_l, m_r)
    ex = jnp.where(valid, jnp.exp(logits - m), 0.0)
    s_l = jnp.sum(jnp.where(left, ex, 0.0), axis=1, keepdims=True)
    s_r = jnp.sum(jnp.where(left, 0.0, ex), axis=1, keepdims=True)
    lse = jnp.log(jnp.where(left, s_l, s_r)) + m
    out_ref[...] = logits - lse


def _row_spec(f):
    return pl.BlockSpec((_BLK, f), lambda i: (i, 0))


def _full_spec(shape):
    return pl.BlockSpec(shape, lambda i: tuple(0 for _ in shape))


def _tc_prep(deg2, x2, w1d, rex):
    return pl.pallas_call(
        _prep_body,
        grid=(_GRID,),
        in_specs=[pl.BlockSpec((2, _BLK, 32), lambda i: (0, i, 0)),
                  _row_spec(2 * F_IN), _full_spec((2 * F_IN, 128)),
                  _full_spec((32, 128))],
        out_specs=[_row_spec(128), _row_spec(128)],
        out_shape=[jax.ShapeDtypeStruct((NP2, 128), jnp.float32),
                   jax.ShapeDtypeStruct((NP2, 128), jnp.float32)],
    )(deg2, x2, w1d, rex)


def _tc_combine(acc2, hp, dis, b, wd):
    return pl.pallas_call(
        _combine_body,
        grid=(_GRID,),
        in_specs=[pl.BlockSpec((2, _BLK, 128), lambda i: (0, i, 0)),
                  _row_spec(128), _row_spec(128), _full_spec((1, 128)),
                  _full_spec((128, 128))],
        out_specs=[_row_spec(128)],
        out_shape=[jax.ShapeDtypeStruct((NP2, 128), jnp.float32)],
    )(acc2, hp, dis, b, wd)[0]


def _tc_final(acc2, hp, dis, b):
    return pl.pallas_call(
        _final_body,
        grid=(_GRID,),
        in_specs=[pl.BlockSpec((2, _BLK, 128), lambda i: (0, i, 0)),
                  _row_spec(128), _row_spec(128), _full_spec((1, 128))],
        out_specs=[_row_spec(128)],
        out_shape=[jax.ShapeDtypeStruct((NP2, 128), jnp.float32)],
    )(acc2, hp, dis, b)[0]


def _blockdiag(w):
    fi, fo = w.shape
    z = jnp.zeros((fi, fo), jnp.float32)
    return jnp.concatenate([
        jnp.concatenate([w, z], axis=1),
        jnp.concatenate([z, w], axis=1),
    ], axis=0)


def kernel(x, edge_index, W1, b1, W2, b2, W3, b3):
    ones16 = jnp.ones((K, 16), jnp.float32)
    z16 = jnp.zeros((NP, 16), jnp.float32)
    z64 = jnp.zeros((NP, FP), jnp.float32)
    x2 = jnp.pad(x, ((0, NP - N), (0, 0))).reshape(NP2, 2 * F_IN)
    w3p = jnp.pad(W3, ((0, 0), (0, FP - C)))
    w1d = _blockdiag(W1)
    w2d = _blockdiag(W2)
    w3d = _blockdiag(w3p)
    rex = jnp.zeros((32, 128), jnp.float32)
    rex = rex.at[0, :64].set(1.0).at[16, 64:].set(1.0)
    b1p = jnp.concatenate([b1, b1]).reshape(1, 128)
    b2p = jnp.concatenate([b2, b2]).reshape(1, 128)
    b3f = jnp.pad(b3, (0, FP - C))
    b3p = jnp.concatenate([b3f, b3f]).reshape(1, 128)

    eif = edge_index.reshape(2 * E)
    deg = _sc_degree(eif, ones16, z16)
    deg2 = deg.reshape(NC, NP2, 32)
    dis, hp1 = _tc_prep(deg2, x2, w1d, rex)

    acc1 = _sc_msg(hp1.reshape(NP, FP), eif, z64)
    hp2 = _tc_combine(acc1.reshape(NC, NP2, 128), hp1, dis, b1p, w2d)

    acc2 = _sc_msg(hp2.reshape(NP, FP), eif, z64)
    hp3 = _tc_combine(acc2.reshape(NC, NP2, 128), hp2, dis, b2p, w3d)

    acc3 = _sc_msg(hp3.reshape(NP, FP), eif, z64)
    out = _tc_final(acc3.reshape(NC, NP2, 128), hp3, dis, b3p)
    return out.reshape(NP, FP)[:N, :C]


# speedup vs baseline: 1.2661x; 1.0163x over previous
import functools

import jax
import jax.numpy as jnp
from jax import lax
from jax.experimental import pallas as pl
from jax.experimental.pallas import tpu as pltpu
from jax.experimental.pallas import tpu_sc as plsc

N = 10000
NP = 10240
E = 320000
F_IN = 128
H = 64
C = 40
FP = 64

NC = 2
NS = 16
NW = NC * NS
EPT = E // NW
K = 400
NCHUNK = EPT // K
RPT = NP // NS

_mesh = plsc.VectorSubcoreMesh(core_axis_name="c", subcore_axis_name="s")


@functools.partial(
    pl.kernel,
    out_type=jax.ShapeDtypeStruct((NC, NP, 16), jnp.float32),
    mesh=_mesh,
    compiler_params=pltpu.CompilerParams(use_tc_tiling_on_sc=False),
    scratch_types=[
        pltpu.VMEM((EPT,), jnp.int32),
        pltpu.VMEM((K, 16), jnp.float32),
        pltpu.VMEM_SHARED((NP, 16), jnp.float32),
        pltpu.SemaphoreType.DMA,
    ],
)
def _sc_degree(ei_hbm, ones_hbm, z16_hbm, out_hbm, colv, onesv, acc, sem):
    c = lax.axis_index("c")
    s = lax.axis_index("s")
    wid = c * NS + s
    pltpu.sync_copy(z16_hbm.at[pl.ds(s * RPT, RPT)], acc.at[pl.ds(s * RPT, RPT)])
    pltpu.sync_copy(ones_hbm, onesv)
    pltpu.sync_copy(ei_hbm.at[pl.ds(E + wid * EPT, EPT)], colv)
    plsc.subcore_barrier()

    @pl.loop(0, NCHUNK, step=5)
    def _group(g):
        for b in range(5):
            pltpu.async_copy(onesv, acc.at[colv.at[pl.ds((g + b) * K, K)]],
                             sem, add=True)
        for b in range(5):
            pltpu.make_async_copy(onesv, acc.at[colv.at[pl.ds((g + b) * K, K)]],
                                  sem).wait()

    plsc.subcore_barrier()
    pltpu.sync_copy(acc.at[pl.ds(s * RPT, RPT)], out_hbm.at[c, pl.ds(s * RPT, RPT)])


@functools.partial(
    pl.kernel,
    out_type=jax.ShapeDtypeStruct((NC, NP, FP), jnp.float32),
    mesh=_mesh,
    compiler_params=pltpu.CompilerParams(use_tc_tiling_on_sc=False),
    scratch_types=[
        pltpu.VMEM((EPT,), jnp.int32),
        pltpu.VMEM((EPT,), jnp.int32),
        pltpu.VMEM((K, FP), jnp.float32),
        pltpu.VMEM((K, FP), jnp.float32),
        pltpu.VMEM_SHARED((NP, FP), jnp.float32),
        pltpu.SemaphoreType.DMA,
        pltpu.SemaphoreType.DMA,
    ],
)
def _sc_msg(hp_hbm, ei_hbm, z64_hbm, out_hbm,
            rowv, colv, buf_a, buf_b, acc, sem_a, sem_b):
    c = lax.axis_index("c")
    s = lax.axis_index("s")
    wid = c * NS + s
    pltpu.sync_copy(z64_hbm.at[pl.ds(s * RPT, RPT)], acc.at[pl.ds(s * RPT, RPT)])
    pltpu.sync_copy(ei_hbm.at[pl.ds(wid * EPT, EPT)], rowv)
    pltpu.sync_copy(ei_hbm.at[pl.ds(E + wid * EPT, EPT)], colv)
    plsc.subcore_barrier()

    def _gather(j, buf, sem):
        pltpu.async_copy(hp_hbm.at[rowv.at[pl.ds(j * K, K)]], buf, sem)

    def _gather_wait(j, buf, sem):
        pltpu.make_async_copy(hp_hbm.at[rowv.at[pl.ds(j * K, K)]], buf, sem).wait()

    KH = K // 2

    def _scatter_sync(j, buf):
        pltpu.sync_copy(buf.at[pl.ds(0, KH)],
                        acc.at[colv.at[pl.ds(j * K, KH)]], add=True)
        pltpu.sync_copy(buf.at[pl.ds(KH, KH)],
                        acc.at[colv.at[pl.ds(j * K + KH, KH)]], add=True)

    _gather(0, buf_a, sem_a)

    @pl.loop(0, NCHUNK - 1, step=2)
    def _pair(j):
        _gather(j + 1, buf_b, sem_b)
        _gather_wait(j, buf_a, sem_a)
        _scatter_sync(j, buf_a)

        @pl.when(j + 2 < NCHUNK)
        def _():
            _gather(j + 2, buf_a, sem_a)

        _gather_wait(j + 1, buf_b, sem_b)
        _scatter_sync(j + 1, buf_b)

    if NCHUNK % 2:
        _gather_wait(NCHUNK - 1, buf_a, sem_a)
        _scatter_sync(NCHUNK - 1, buf_a)

    plsc.subcore_barrier()
    pltpu.sync_copy(acc.at[pl.ds(s * RPT, RPT)], out_hbm.at[c, pl.ds(s * RPT, RPT)])


NP2 = NP // 2
_BLK = 2560
_GRID = NP2 // _BLK


def _prep_body(deg_ref, x2_ref, w1d_ref, rex_ref, dis_ref, hp_ref):
    d32 = 1.0 + deg_ref[0] + deg_ref[1]
    dis32 = lax.rsqrt(d32)
    dis = jnp.dot(dis32, rex_ref[...], preferred_element_type=jnp.float32)
    dis_ref[...] = dis
    h = jnp.dot(x2_ref[...], w1d_ref[...], preferred_element_type=jnp.float32,
                precision=lax.Precision.HIGHEST)
    hp_ref[...] = dis * h


def _combine_body(acc_ref, hp_ref, dis_ref, b_ref, wd_ref, hpn_ref):
    dis = dis_ref[...]
    h = dis * (acc_ref[0] + acc_ref[1] + hp_ref[...]) + b_ref[...]
    h = jnp.maximum(h, 0.0)
    hn = jnp.dot(h, wd_ref[...], preferred_element_type=jnp.float32,
                 precision=lax.Precision.HIGHEST)
    hpn_ref[...] = dis * hn


def _final_body(acc_ref, hp_ref, dis_ref, b_ref, out_ref):
    logits = dis_ref[...] * (acc_ref[0] + acc_ref[1] + hp_ref[...]) + b_ref[...]
    lanes = lax.broadcasted_iota(jnp.int32, logits.shape, 1)
    left = lanes < FP
    valid = (lanes % FP) < C
    neg = jnp.float32(-jnp.inf)
    lm = jnp.where(valid, logits, neg)
    m_l = jnp.max(jnp.where(left, lm, neg), axis=1, keepdims=True)
    m_r = jnp.max(jnp.where(left, neg, lm), axis=1, keepdims=True)
    m = jnp.where(left, m_l, m_r)
    ex = jnp.where(valid, jnp.exp(logits - m), 0.0)
    s_l = jnp.sum(jnp.where(left, ex, 0.0), axis=1, keepdims=True)
    s_r = jnp.sum(jnp.where(left, 0.0, ex), axis=1, keepdims=True)
    lse = jnp.log(jnp.where(left, s_l, s_r)) + m
    out_ref[...] = logits - lse


def _row_spec(f):
    return pl.BlockSpec((_BLK, f), lambda i: (i, 0))


def _full_spec(shape):
    return pl.BlockSpec(shape, lambda i: tuple(0 for _ in shape))


def _tc_prep(deg2, x2, w1d, rex):
    return pl.pallas_call(
        _prep_body,
        grid=(_GRID,),
        in_specs=[pl.BlockSpec((2, _BLK, 32), lambda i: (0, i, 0)),
                  _row_spec(2 * F_IN), _full_spec((2 * F_IN, 128)),
                  _full_spec((32, 128))],
        out_specs=[_row_spec(128), _row_spec(128)],
        out_shape=[jax.ShapeDtypeStruct((NP2, 128), jnp.float32),
                   jax.ShapeDtypeStruct((NP2, 128), jnp.float32)],
    )(deg2, x2, w1d, rex)


def _tc_combine(acc2, hp, dis, b, wd):
    return pl.pallas_call(
        _combine_body,
        grid=(_GRID,),
        in_specs=[pl.BlockSpec((2, _BLK, 128), lambda i: (0, i, 0)),
                  _row_spec(128), _row_spec(128), _full_spec((1, 128)),
                  _full_spec((128, 128))],
        out_specs=[_row_spec(128)],
        out_shape=[jax.ShapeDtypeStruct((NP2, 128), jnp.float32)],
    )(acc2, hp, dis, b, wd)[0]


def _tc_final(acc2, hp, dis, b):
    return pl.pallas_call(
        _final_body,
        grid=(_GRID,),
        in_specs=[pl.BlockSpec((2, _BLK, 128), lambda i: (0, i, 0)),
                  _row_spec(128), _row_spec(128), _full_spec((1, 128))],
        out_specs=[_row_spec(128)],
        out_shape=[jax.ShapeDtypeStruct((NP2, 128), jnp.float32)],
    )(acc2, hp, dis, b)[0]


def _blockdiag(w):
    fi, fo = w.shape
    z = jnp.zeros((fi, fo), jnp.float32)
    return jnp.concatenate([
        jnp.concatenate([w, z], axis=1),
        jnp.concatenate([z, w], axis=1),
    ], axis=0)


def kernel(x, edge_index, W1, b1, W2, b2, W3, b3):
    ones16 = jnp.ones((K, 16), jnp.float32)
    z16 = jnp.zeros((NP, 16), jnp.float32)
    z64 = jnp.zeros((NP, FP), jnp.float32)
    x2 = jnp.pad(x, ((0, NP - N), (0, 0))).reshape(NP2, 2 * F_IN)
    w3p = jnp.pad(W3, ((0, 0), (0, FP - C)))
    w1d = _blockdiag(W1)
    w2d = _blockdiag(W2)
    w3d = _blockdiag(w3p)
    rex = jnp.zeros((32, 128), jnp.float32)
    rex = rex.at[0, :64].set(1.0).at[16, 64:].set(1.0)
    b1p = jnp.concatenate([b1, b1]).reshape(1, 128)
    b2p = jnp.concatenate([b2, b2]).reshape(1, 128)
    b3f = jnp.pad(b3, (0, FP - C))
    b3p = jnp.concatenate([b3f, b3f]).reshape(1, 128)

    eif = edge_index.reshape(2 * E)
    deg = _sc_degree(eif, ones16, z16)
    deg2 = deg.reshape(NC, NP2, 32)
    dis, hp1 = _tc_prep(deg2, x2, w1d, rex)

    acc1 = _sc_msg(hp1.reshape(NP, FP), eif, z64)
    hp2 = _tc_combine(acc1.reshape(NC, NP2, 128), hp1, dis, b1p, w2d)

    acc2 = _sc_msg(hp2.reshape(NP, FP), eif, z64)
    hp3 = _tc_combine(acc2.reshape(NC, NP2, 128), hp2, dis, b2p, w3d)

    acc3 = _sc_msg(hp3.reshape(NP, FP), eif, z64)
    out = _tc_final(acc3.reshape(NC, NP2, 128), hp3, dis, b3p)
    return out.reshape(NP, FP)[:N, :C]


# speedup vs baseline: 1.2692x; 1.0025x over previous
import functools

import jax
import jax.numpy as jnp
from jax import lax
from jax.experimental import pallas as pl
from jax.experimental.pallas import tpu as pltpu
from jax.experimental.pallas import tpu_sc as plsc

N = 10000
NP = 10240
E = 320000
F_IN = 128
H = 64
C = 40
FP = 64

NC = 2
NS = 16
NW = NC * NS
EPT = E // NW
K = 400
NCHUNK = EPT // K
RPT = NP // NS

_mesh = plsc.VectorSubcoreMesh(core_axis_name="c", subcore_axis_name="s")


@functools.partial(
    pl.kernel,
    out_type=jax.ShapeDtypeStruct((NC, NP, 16), jnp.float32),
    mesh=_mesh,
    compiler_params=pltpu.CompilerParams(use_tc_tiling_on_sc=False),
    scratch_types=[
        pltpu.VMEM((EPT,), jnp.int32),
        pltpu.VMEM((K, 16), jnp.float32),
        pltpu.VMEM_SHARED((NP, 16), jnp.float32),
        pltpu.SemaphoreType.DMA,
    ],
)
def _sc_degree(ei_hbm, ones_hbm, z16_hbm, out_hbm, colv, onesv, acc, sem):
    c = lax.axis_index("c")
    s = lax.axis_index("s")
    wid = c * NS + s
    pltpu.sync_copy(z16_hbm.at[pl.ds(s * RPT, RPT)], acc.at[pl.ds(s * RPT, RPT)])
    pltpu.sync_copy(ones_hbm, onesv)
    pltpu.sync_copy(ei_hbm.at[pl.ds(E + wid * EPT, EPT)], colv)
    plsc.subcore_barrier()

    @pl.loop(0, NCHUNK, step=5)
    def _group(g):
        for b in range(5):
            for h in range(2):
                pltpu.async_copy(
                    onesv.at[pl.ds(0, K // 2)],
                    acc.at[colv.at[pl.ds((g + b) * K + h * (K // 2), K // 2)]],
                    sem, add=True)
        for b in range(5):
            for h in range(2):
                pltpu.make_async_copy(
                    onesv.at[pl.ds(0, K // 2)],
                    acc.at[colv.at[pl.ds((g + b) * K + h * (K // 2), K // 2)]],
                    sem).wait()

    plsc.subcore_barrier()
    pltpu.sync_copy(acc.at[pl.ds(s * RPT, RPT)], out_hbm.at[c, pl.ds(s * RPT, RPT)])


@functools.partial(
    pl.kernel,
    out_type=jax.ShapeDtypeStruct((NC, NP, FP), jnp.float32),
    mesh=_mesh,
    compiler_params=pltpu.CompilerParams(use_tc_tiling_on_sc=False),
    scratch_types=[
        pltpu.VMEM((EPT,), jnp.int32),
        pltpu.VMEM((EPT,), jnp.int32),
        pltpu.VMEM((K, FP), jnp.float32),
        pltpu.VMEM((K, FP), jnp.float32),
        pltpu.VMEM_SHARED((NP, FP), jnp.float32),
        pltpu.SemaphoreType.DMA,
        pltpu.SemaphoreType.DMA,
    ],
)
def _sc_msg(hp_hbm, ei_hbm, z64_hbm, out_hbm,
            rowv, colv, buf_a, buf_b, acc, sem_a, sem_b):
    c = lax.axis_index("c")
    s = lax.axis_index("s")
    wid = c * NS + s
    pltpu.sync_copy(z64_hbm.at[pl.ds(s * RPT, RPT)], acc.at[pl.ds(s * RPT, RPT)])
    pltpu.sync_copy(ei_hbm.at[pl.ds(wid * EPT, EPT)], rowv)
    pltpu.sync_copy(ei_hbm.at[pl.ds(E + wid * EPT, EPT)], colv)
    plsc.subcore_barrier()

    def _gather(j, buf, sem):
        pltpu.async_copy(hp_hbm.at[rowv.at[pl.ds(j * K, K)]], buf, sem)

    def _gather_wait(j, buf, sem):
        pltpu.make_async_copy(hp_hbm.at[rowv.at[pl.ds(j * K, K)]], buf, sem).wait()

    KH = K // 2

    def _scatter_sync(j, buf):
        pltpu.sync_copy(buf.at[pl.ds(0, KH)],
                        acc.at[colv.at[pl.ds(j * K, KH)]], add=True)
        pltpu.sync_copy(buf.at[pl.ds(KH, KH)],
                        acc.at[colv.at[pl.ds(j * K + KH, KH)]], add=True)

    _gather(0, buf_a, sem_a)

    @pl.loop(0, NCHUNK - 1, step=2)
    def _pair(j):
        _gather(j + 1, buf_b, sem_b)
        _gather_wait(j, buf_a, sem_a)
        _scatter_sync(j, buf_a)

        @pl.when(j + 2 < NCHUNK)
        def _():
            _gather(j + 2, buf_a, sem_a)

        _gather_wait(j + 1, buf_b, sem_b)
        _scatter_sync(j + 1, buf_b)

    if NCHUNK % 2:
        _gather_wait(NCHUNK - 1, buf_a, sem_a)
        _scatter_sync(NCHUNK - 1, buf_a)

    plsc.subcore_barrier()
    pltpu.sync_copy(acc.at[pl.ds(s * RPT, RPT)], out_hbm.at[c, pl.ds(s * RPT, RPT)])


NP2 = NP // 2
_BLK = 2560
_GRID = NP2 // _BLK


def _prep_body(deg_ref, x2_ref, w1d_ref, rex_ref, dis_ref, hp_ref):
    d32 = 1.0 + deg_ref[0] + deg_ref[1]
    dis32 = lax.rsqrt(d32)
    dis = jnp.dot(dis32, rex_ref[...], preferred_element_type=jnp.float32)
    dis_ref[...] = dis
    h = jnp.dot(x2_ref[...], w1d_ref[...], preferred_element_type=jnp.float32,
                precision=lax.Precision.HIGHEST)
    hp_ref[...] = dis * h


def _combine_body(acc_ref, hp_ref, dis_ref, b_ref, wd_ref, hpn_ref):
    dis = dis_ref[...]
    h = dis * (acc_ref[0] + acc_ref[1] + hp_ref[...]) + b_ref[...]
    h = jnp.maximum(h, 0.0)
    hn = jnp.dot(h, wd_ref[...], preferred_element_type=jnp.float32,
                 precision=lax.Precision.HIGHEST)
    hpn_ref[...] = dis * hn


def _final_body(acc_ref, hp_ref, dis_ref, b_ref, out_ref):
    logits = dis_ref[...] * (acc_ref[0] + acc_ref[1] + hp_ref[...]) + b_ref[...]
    lanes = lax.broadcasted_iota(jnp.int32, logits.shape, 1)
    left = lanes < FP
    valid = (lanes % FP) < C
    neg = jnp.float32(-jnp.inf)
    lm = jnp.where(valid, logits, neg)
    m_l = jnp.max(jnp.where(left, lm, neg), axis=1, keepdims=True)
    m_r = jnp.max(jnp.where(left, neg, lm), axis=1, keepdims=True)
    m = jnp.where(left, m_l, m_r)
    ex = jnp.where(valid, jnp.exp(logits - m), 0.0)
    s_l = jnp.sum(jnp.where(left, ex, 0.0), axis=1, keepdims=True)
    s_r = jnp.sum(jnp.where(left, 0.0, ex), axis=1, keepdims=True)
    lse = jnp.log(jnp.where(left, s_l, s_r)) + m
    out_ref[...] = logits - lse


def _row_spec(f):
    return pl.BlockSpec((_BLK, f), lambda i: (i, 0))


def _full_spec(shape):
    return pl.BlockSpec(shape, lambda i: tuple(0 for _ in shape))


def _tc_prep(deg2, x2, w1d, rex):
    return pl.pallas_call(
        _prep_body,
        grid=(_GRID,),
        in_specs=[pl.BlockSpec((2, _BLK, 32), lambda i: (0, i, 0)),
                  _row_spec(2 * F_IN), _full_spec((2 * F_IN, 128)),
                  _full_spec((32, 128))],
        out_specs=[_row_spec(128), _row_spec(128)],
        out_shape=[jax.ShapeDtypeStruct((NP2, 128), jnp.float32),
                   jax.ShapeDtypeStruct((NP2, 128), jnp.float32)],
    )(deg2, x2, w1d, rex)


def _tc_combine(acc2, hp, dis, b, wd):
    return pl.pallas_call(
        _combine_body,
        grid=(_GRID,),
        in_specs=[pl.BlockSpec((2, _BLK, 128), lambda i: (0, i, 0)),
                  _row_spec(128), _row_spec(128), _full_spec((1, 128)),
                  _full_spec((128, 128))],
        out_specs=[_row_spec(128)],
        out_shape=[jax.ShapeDtypeStruct((NP2, 128), jnp.float32)],
    )(acc2, hp, dis, b, wd)[0]


def _tc_final(acc2, hp, dis, b):
    return pl.pallas_call(
        _final_body,
        grid=(_GRID,),
        in_specs=[pl.BlockSpec((2, _BLK, 128), lambda i: (0, i, 0)),
                  _row_spec(128), _row_spec(128), _full_spec((1, 128))],
        out_specs=[_row_spec(128)],
        out_shape=[jax.ShapeDtypeStruct((NP2, 128), jnp.float32)],
    )(acc2, hp, dis, b)[0]


def _blockdiag(w):
    fi, fo = w.shape
    z = jnp.zeros((fi, fo), jnp.float32)
    return jnp.concatenate([
        jnp.concatenate([w, z], axis=1),
        jnp.concatenate([z, w], axis=1),
    ], axis=0)


def kernel(x, edge_index, W1, b1, W2, b2, W3, b3):
    ones16 = jnp.ones((K, 16), jnp.float32)
    z16 = jnp.zeros((NP, 16), jnp.float32)
    z64 = jnp.zeros((NP, FP), jnp.float32)
    x2 = jnp.pad(x, ((0, NP - N), (0, 0))).reshape(NP2, 2 * F_IN)
    w3p = jnp.pad(W3, ((0, 0), (0, FP - C)))
    w1d = _blockdiag(W1)
    w2d = _blockdiag(W2)
    w3d = _blockdiag(w3p)
    rex = jnp.zeros((32, 128), jnp.float32)
    rex = rex.at[0, :64].set(1.0).at[16, 64:].set(1.0)
    b1p = jnp.concatenate([b1, b1]).reshape(1, 128)
    b2p = jnp.concatenate([b2, b2]).reshape(1, 128)
    b3f = jnp.pad(b3, (0, FP - C))
    b3p = jnp.concatenate([b3f, b3f]).reshape(1, 128)

    eif = edge_index.reshape(2 * E)
    deg = _sc_degree(eif, ones16, z16)
    deg2 = deg.reshape(NC, NP2, 32)
    dis, hp1 = _tc_prep(deg2, x2, w1d, rex)

    acc1 = _sc_msg(hp1.reshape(NP, FP), eif, z64)
    hp2 = _tc_combine(acc1.reshape(NC, NP2, 128), hp1, dis, b1p, w2d)

    acc2 = _sc_msg(hp2.reshape(NP, FP), eif, z64)
    hp3 = _tc_combine(acc2.reshape(NC, NP2, 128), hp2, dis, b2p, w3d)

    acc3 = _sc_msg(hp3.reshape(NP, FP), eif, z64)
    out = _tc_final(acc3.reshape(NC, NP2, 128), hp3, dis, b3p)
    return out.reshape(NP, FP)[:N, :C]


# speedup vs baseline: 1.3285x; 1.0467x over previous
import functools

import jax
import jax.numpy as jnp
from jax import lax
from jax.experimental import pallas as pl
from jax.experimental.pallas import tpu as pltpu
from jax.experimental.pallas import tpu_sc as plsc

N = 10000
NP = 10240
E = 320000
F_IN = 128
H = 64
C = 40
FP = 64

NC = 2
NS = 16
NW = NC * NS
EPT = E // NW
K = 400
NCHUNK = EPT // K
RPT = NP // NS

_mesh = plsc.VectorSubcoreMesh(core_axis_name="c", subcore_axis_name="s")


@functools.partial(
    pl.kernel,
    out_type=jax.ShapeDtypeStruct((NC, NP, 16), jnp.float32),
    mesh=_mesh,
    compiler_params=pltpu.CompilerParams(use_tc_tiling_on_sc=False),
    scratch_types=[
        pltpu.VMEM((EPT,), jnp.int32),
        pltpu.VMEM((K, 16), jnp.float32),
        pltpu.VMEM_SHARED((NP, 16), jnp.float32),
        pltpu.SemaphoreType.DMA,
    ],
)
def _sc_degree(ei_hbm, ones_hbm, z16_hbm, out_hbm, colv, onesv, acc, sem):
    c = lax.axis_index("c")
    s = lax.axis_index("s")
    wid = c * NS + s
    pltpu.sync_copy(z16_hbm.at[pl.ds(s * RPT, RPT)], acc.at[pl.ds(s * RPT, RPT)])
    pltpu.sync_copy(ones_hbm, onesv)
    pltpu.sync_copy(ei_hbm.at[pl.ds(E + wid * EPT, EPT)], colv)
    plsc.subcore_barrier()

    @pl.loop(0, NCHUNK, step=5)
    def _group(g):
        for b in range(5):
            for h in range(2):
                pltpu.async_copy(
                    onesv.at[pl.ds(0, K // 2)],
                    acc.at[colv.at[pl.ds((g + b) * K + h * (K // 2), K // 2)]],
                    sem, add=True)
        for b in range(5):
            for h in range(2):
                pltpu.make_async_copy(
                    onesv.at[pl.ds(0, K // 2)],
                    acc.at[colv.at[pl.ds((g + b) * K + h * (K // 2), K // 2)]],
                    sem).wait()

    plsc.subcore_barrier()
    pltpu.sync_copy(acc.at[pl.ds(s * RPT, RPT)], out_hbm.at[c, pl.ds(s * RPT, RPT)])


@functools.partial(
    pl.kernel,
    out_type=jax.ShapeDtypeStruct((NC, NP, FP), jnp.float32),
    mesh=_mesh,
    compiler_params=pltpu.CompilerParams(use_tc_tiling_on_sc=False),
    scratch_types=[
        pltpu.VMEM((EPT,), jnp.int32),
        pltpu.VMEM((EPT,), jnp.int32),
        pltpu.VMEM((K, FP), jnp.float32),
        pltpu.VMEM((K, FP), jnp.float32),
        pltpu.VMEM_SHARED((NP, FP), jnp.float32),
        pltpu.SemaphoreType.DMA,
        pltpu.SemaphoreType.DMA,
    ],
)
def _sc_msg(hp_hbm, ei_hbm, z64_hbm, out_hbm,
            rowv, colv, buf_a, buf_b, acc, sem_a, sem_b):
    c = lax.axis_index("c")
    s = lax.axis_index("s")
    wid = c * NS + s
    pltpu.async_copy(z64_hbm.at[pl.ds(s * RPT, RPT)],
                     acc.at[pl.ds(s * RPT, RPT)], sem_b)
    pltpu.async_copy(ei_hbm.at[pl.ds(wid * EPT, EPT)], rowv, sem_a)
    pltpu.make_async_copy(ei_hbm.at[pl.ds(wid * EPT, EPT)], rowv, sem_a).wait()

    def _gather(j, buf, sem):
        pltpu.async_copy(hp_hbm.at[rowv.at[pl.ds(j * K, K)]], buf, sem)

    def _gather_wait(j, buf, sem):
        pltpu.make_async_copy(hp_hbm.at[rowv.at[pl.ds(j * K, K)]], buf, sem).wait()

    KH = K // 2

    def _scatter_sync(j, buf):
        pltpu.sync_copy(buf.at[pl.ds(0, KH)],
                        acc.at[colv.at[pl.ds(j * K, KH)]], add=True)
        pltpu.sync_copy(buf.at[pl.ds(KH, KH)],
                        acc.at[colv.at[pl.ds(j * K + KH, KH)]], add=True)

    _gather(0, buf_a, sem_a)
    pltpu.sync_copy(ei_hbm.at[pl.ds(E + wid * EPT, EPT)], colv)
    pltpu.make_async_copy(z64_hbm.at[pl.ds(s * RPT, RPT)],
                          acc.at[pl.ds(s * RPT, RPT)], sem_b).wait()
    plsc.subcore_barrier()

    @pl.loop(0, NCHUNK - 1, step=2)
    def _pair(j):
        _gather(j + 1, buf_b, sem_b)
        _gather_wait(j, buf_a, sem_a)
        _scatter_sync(j, buf_a)

        @pl.when(j + 2 < NCHUNK)
        def _():
            _gather(j + 2, buf_a, sem_a)

        _gather_wait(j + 1, buf_b, sem_b)
        _scatter_sync(j + 1, buf_b)

    if NCHUNK % 2:
        _gather_wait(NCHUNK - 1, buf_a, sem_a)
        _scatter_sync(NCHUNK - 1, buf_a)

    plsc.subcore_barrier()
    pltpu.sync_copy(acc.at[pl.ds(s * RPT, RPT)], out_hbm.at[c, pl.ds(s * RPT, RPT)])


NP2 = NP // 2
_BLK = 2560
_GRID = NP2 // _BLK


def _prep_body(deg_ref, x2_ref, w1d_ref, rex_ref, dis_ref, hp_ref):
    d32 = 1.0 + deg_ref[0] + deg_ref[1]
    dis32 = lax.rsqrt(d32)
    dis = jnp.dot(dis32, rex_ref[...], preferred_element_type=jnp.float32)
    dis_ref[...] = dis
    h = jnp.dot(x2_ref[...], w1d_ref[...], preferred_element_type=jnp.float32,
                precision=lax.Precision.HIGHEST)
    hp_ref[...] = dis * h


def _combine_body(acc_ref, hp_ref, dis_ref, b_ref, wd_ref, hpn_ref):
    dis = dis_ref[...]
    h = dis * (acc_ref[0] + acc_ref[1] + hp_ref[...]) + b_ref[...]
    h = jnp.maximum(h, 0.0)
    hn = jnp.dot(h, wd_ref[...], preferred_element_type=jnp.float32,
                 precision=lax.Precision.HIGHEST)
    hpn_ref[...] = dis * hn


def _final_body(acc_ref, hp_ref, dis_ref, b_ref, out_ref):
    logits = dis_ref[...] * (acc_ref[0] + acc_ref[1] + hp_ref[...]) + b_ref[...]
    lanes = lax.broadcasted_iota(jnp.int32, logits.shape, 1)
    left = lanes < FP
    valid = (lanes % FP) < C
    neg = jnp.float32(-jnp.inf)
    lm = jnp.where(valid, logits, neg)
    m_l = jnp.max(jnp.where(left, lm, neg), axis=1, keepdims=True)
    m_r = jnp.max(jnp.where(left, neg, lm), axis=1, keepdims=True)
    m = jnp.where(left, m_l, m_r)
    ex = jnp.where(valid, jnp.exp(logits - m), 0.0)
    s_l = jnp.sum(jnp.where(left, ex, 0.0), axis=1, keepdims=True)
    s_r = jnp.sum(jnp.where(left, 0.0, ex), axis=1, keepdims=True)
    lse = jnp.log(jnp.where(left, s_l, s_r)) + m
    out_ref[...] = logits - lse


def _row_spec(f):
    return pl.BlockSpec((_BLK, f), lambda i: (i, 0))


def _full_spec(shape):
    return pl.BlockSpec(shape, lambda i: tuple(0 for _ in shape))


def _tc_prep(deg2, x2, w1d, rex):
    return pl.pallas_call(
        _prep_body,
        grid=(_GRID,),
        in_specs=[pl.BlockSpec((2, _BLK, 32), lambda i: (0, i, 0)),
                  _row_spec(2 * F_IN), _full_spec((2 * F_IN, 128)),
                  _full_spec((32, 128))],
        out_specs=[_row_spec(128), _row_spec(128)],
        out_shape=[jax.ShapeDtypeStruct((NP2, 128), jnp.float32),
                   jax.ShapeDtypeStruct((NP2, 128), jnp.float32)],
    )(deg2, x2, w1d, rex)


def _tc_combine(acc2, hp, dis, b, wd):
    return pl.pallas_call(
        _combine_body,
        grid=(_GRID,),
        in_specs=[pl.BlockSpec((2, _BLK, 128), lambda i: (0, i, 0)),
                  _row_spec(128), _row_spec(128), _full_spec((1, 128)),
                  _full_spec((128, 128))],
        out_specs=[_row_spec(128)],
        out_shape=[jax.ShapeDtypeStruct((NP2, 128), jnp.float32)],
    )(acc2, hp, dis, b, wd)[0]


def _tc_final(acc2, hp, dis, b):
    return pl.pallas_call(
        _final_body,
        grid=(_GRID,),
        in_specs=[pl.BlockSpec((2, _BLK, 128), lambda i: (0, i, 0)),
                  _row_spec(128), _row_spec(128), _full_spec((1, 128))],
        out_specs=[_row_spec(128)],
        out_shape=[jax.ShapeDtypeStruct((NP2, 128), jnp.float32)],
    )(acc2, hp, dis, b)[0]


def _blockdiag(w):
    fi, fo = w.shape
    z = jnp.zeros((fi, fo), jnp.float32)
    return jnp.concatenate([
        jnp.concatenate([w, z], axis=1),
        jnp.concatenate([z, w], axis=1),
    ], axis=0)


def kernel(x, edge_index, W1, b1, W2, b2, W3, b3):
    ones16 = jnp.ones((K, 16), jnp.float32)
    z16 = jnp.zeros((NP, 16), jnp.float32)
    z64 = jnp.zeros((NP, FP), jnp.float32)
    x2 = jnp.pad(x, ((0, NP - N), (0, 0))).reshape(NP2, 2 * F_IN)
    w3p = jnp.pad(W3, ((0, 0), (0, FP - C)))
    w1d = _blockdiag(W1)
    w2d = _blockdiag(W2)
    w3d = _blockdiag(w3p)
    rex = jnp.zeros((32, 128), jnp.float32)
    rex = rex.at[0, :64].set(1.0).at[16, 64:].set(1.0)
    b1p = jnp.concatenate([b1, b1]).reshape(1, 128)
    b2p = jnp.concatenate([b2, b2]).reshape(1, 128)
    b3f = jnp.pad(b3, (0, FP - C))
    b3p = jnp.concatenate([b3f, b3f]).reshape(1, 128)

    eif = edge_index.reshape(2 * E)
    deg = _sc_degree(eif, ones16, z16)
    deg2 = deg.reshape(NC, NP2, 32)
    dis, hp1 = _tc_prep(deg2, x2, w1d, rex)

    acc1 = _sc_msg(hp1.reshape(NP, FP), eif, z64)
    hp2 = _tc_combine(acc1.reshape(NC, NP2, 128), hp1, dis, b1p, w2d)

    acc2 = _sc_msg(hp2.reshape(NP, FP), eif, z64)
    hp3 = _tc_combine(acc2.reshape(NC, NP2, 128), hp2, dis, b2p, w3d)

    acc3 = _sc_msg(hp3.reshape(NP, FP), eif, z64)
    out = _tc_final(acc3.reshape(NC, NP2, 128), hp3, dis, b3p)
    return out.reshape(NP, FP)[:N, :C]


# speedup vs baseline: 1.3354x; 1.0052x over previous
import functools

import jax
import jax.numpy as jnp
from jax import lax
from jax.experimental import pallas as pl
from jax.experimental.pallas import tpu as pltpu
from jax.experimental.pallas import tpu_sc as plsc

N = 10000
NP = 10240
E = 320000
F_IN = 128
H = 64
C = 40
FP = 64

NC = 2
NS = 16
NW = NC * NS
EPT = E // NW
K = 400
NCHUNK = EPT // K
RPT = NP // NS

_mesh = plsc.VectorSubcoreMesh(core_axis_name="c", subcore_axis_name="s")


@functools.partial(
    pl.kernel,
    out_type=jax.ShapeDtypeStruct((NC, NP, 16), jnp.float32),
    mesh=_mesh,
    compiler_params=pltpu.CompilerParams(use_tc_tiling_on_sc=False),
    scratch_types=[
        pltpu.VMEM((EPT,), jnp.int32),
        pltpu.VMEM((K, 16), jnp.float32),
        pltpu.VMEM_SHARED((NP, 16), jnp.float32),
        pltpu.SemaphoreType.DMA,
    ],
)
def _sc_degree(ei_hbm, ones_hbm, z16_hbm, out_hbm, colv, onesv, acc, sem):
    c = lax.axis_index("c")
    s = lax.axis_index("s")
    wid = c * NS + s
    pltpu.async_copy(z16_hbm.at[pl.ds(s * RPT, RPT)],
                     acc.at[pl.ds(s * RPT, RPT)], sem)
    pltpu.sync_copy(ones_hbm, onesv)
    pltpu.sync_copy(ei_hbm.at[pl.ds(E + wid * EPT, EPT)], colv)
    pltpu.make_async_copy(z16_hbm.at[pl.ds(s * RPT, RPT)],
                          acc.at[pl.ds(s * RPT, RPT)], sem).wait()
    plsc.subcore_barrier()

    @pl.loop(0, NCHUNK, step=5)
    def _group(g):
        for b in range(5):
            for h in range(2):
                pltpu.async_copy(
                    onesv.at[pl.ds(0, K // 2)],
                    acc.at[colv.at[pl.ds((g + b) * K + h * (K // 2), K // 2)]],
                    sem, add=True)
        for b in range(5):
            for h in range(2):
                pltpu.make_async_copy(
                    onesv.at[pl.ds(0, K // 2)],
                    acc.at[colv.at[pl.ds((g + b) * K + h * (K // 2), K // 2)]],
                    sem).wait()

    plsc.subcore_barrier()
    pltpu.sync_copy(acc.at[pl.ds(s * RPT, RPT)], out_hbm.at[c, pl.ds(s * RPT, RPT)])


@functools.partial(
    pl.kernel,
    out_type=jax.ShapeDtypeStruct((NC, NP, FP), jnp.float32),
    mesh=_mesh,
    compiler_params=pltpu.CompilerParams(use_tc_tiling_on_sc=False),
    scratch_types=[
        pltpu.VMEM((EPT,), jnp.int32),
        pltpu.VMEM((EPT,), jnp.int32),
        pltpu.VMEM((K, FP), jnp.float32),
        pltpu.VMEM((K, FP), jnp.float32),
        pltpu.VMEM_SHARED((NP, FP), jnp.float32),
        pltpu.SemaphoreType.DMA,
        pltpu.SemaphoreType.DMA,
    ],
)
def _sc_msg(hp_hbm, ei_hbm, z64_hbm, out_hbm,
            rowv, colv, buf_a, buf_b, acc, sem_a, sem_b):
    c = lax.axis_index("c")
    s = lax.axis_index("s")
    wid = c * NS + s
    pltpu.async_copy(z64_hbm.at[pl.ds(s * RPT, RPT)],
                     acc.at[pl.ds(s * RPT, RPT)], sem_b)
    pltpu.async_copy(ei_hbm.at[pl.ds(wid * EPT, EPT)], rowv, sem_a)
    pltpu.make_async_copy(ei_hbm.at[pl.ds(wid * EPT, EPT)], rowv, sem_a).wait()

    def _gather(j, buf, sem):
        pltpu.async_copy(hp_hbm.at[rowv.at[pl.ds(j * K, K)]], buf, sem)

    def _gather_wait(j, buf, sem):
        pltpu.make_async_copy(hp_hbm.at[rowv.at[pl.ds(j * K, K)]], buf, sem).wait()

    KH = K // 2

    def _scatter_sync(j, buf):
        pltpu.sync_copy(buf.at[pl.ds(0, KH)],
                        acc.at[colv.at[pl.ds(j * K, KH)]], add=True)
        pltpu.sync_copy(buf.at[pl.ds(KH, KH)],
                        acc.at[colv.at[pl.ds(j * K + KH, KH)]], add=True)

    _gather(0, buf_a, sem_a)
    pltpu.sync_copy(ei_hbm.at[pl.ds(E + wid * EPT, EPT)], colv)
    pltpu.make_async_copy(z64_hbm.at[pl.ds(s * RPT, RPT)],
                          acc.at[pl.ds(s * RPT, RPT)], sem_b).wait()
    plsc.subcore_barrier()

    @pl.loop(0, NCHUNK - 1, step=2)
    def _pair(j):
        _gather(j + 1, buf_b, sem_b)
        _gather_wait(j, buf_a, sem_a)
        _scatter_sync(j, buf_a)

        @pl.when(j + 2 < NCHUNK)
        def _():
            _gather(j + 2, buf_a, sem_a)

        _gather_wait(j + 1, buf_b, sem_b)
        _scatter_sync(j + 1, buf_b)

    if NCHUNK % 2:
        _gather_wait(NCHUNK - 1, buf_a, sem_a)
        _scatter_sync(NCHUNK - 1, buf_a)

    plsc.subcore_barrier()
    pltpu.sync_copy(acc.at[pl.ds(s * RPT, RPT)], out_hbm.at[c, pl.ds(s * RPT, RPT)])


NP2 = NP // 2
_BLK = 2560
_GRID = NP2 // _BLK


def _prep_body(deg_ref, x2_ref, w1d_ref, rex_ref, dis_ref, hp_ref):
    d32 = 1.0 + deg_ref[0] + deg_ref[1]
    dis32 = lax.rsqrt(d32)
    dis = jnp.dot(dis32, rex_ref[...], preferred_element_type=jnp.float32)
    dis_ref[...] = dis
    h = jnp.dot(x2_ref[...], w1d_ref[...], preferred_element_type=jnp.float32,
                precision=lax.Precision.HIGHEST)
    hp_ref[...] = dis * h


def _combine_body(acc_ref, hp_ref, dis_ref, b_ref, wd_ref, hpn_ref):
    dis = dis_ref[...]
    h = dis * (acc_ref[0] + acc_ref[1] + hp_ref[...]) + b_ref[...]
    h = jnp.maximum(h, 0.0)
    hn = jnp.dot(h, wd_ref[...], preferred_element_type=jnp.float32,
                 precision=lax.Precision.HIGHEST)
    hpn_ref[...] = dis * hn


def _final_body(acc_ref, hp_ref, dis_ref, b_ref, out_ref):
    logits = dis_ref[...] * (acc_ref[0] + acc_ref[1] + hp_ref[...]) + b_ref[...]
    lanes = lax.broadcasted_iota(jnp.int32, logits.shape, 1)
    left = lanes < FP
    valid = (lanes % FP) < C
    neg = jnp.float32(-jnp.inf)
    lm = jnp.where(valid, logits, neg)
    m_l = jnp.max(jnp.where(left, lm, neg), axis=1, keepdims=True)
    m_r = jnp.max(jnp.where(left, neg, lm), axis=1, keepdims=True)
    m = jnp.where(left, m_l, m_r)
    ex = jnp.where(valid, jnp.exp(logits - m), 0.0)
    s_l = jnp.sum(jnp.where(left, ex, 0.0), axis=1, keepdims=True)
    s_r = jnp.sum(jnp.where(left, 0.0, ex), axis=1, keepdims=True)
    lse = jnp.log(jnp.where(left, s_l, s_r)) + m
    out_ref[...] = logits - lse


def _row_spec(f):
    return pl.BlockSpec((_BLK, f), lambda i: (i, 0))


def _full_spec(shape):
    return pl.BlockSpec(shape, lambda i: tuple(0 for _ in shape))


def _tc_prep(deg2, x2, w1d, rex):
    return pl.pallas_call(
        _prep_body,
        grid=(_GRID,),
        in_specs=[pl.BlockSpec((2, _BLK, 32), lambda i: (0, i, 0)),
                  _row_spec(2 * F_IN), _full_spec((2 * F_IN, 128)),
                  _full_spec((32, 128))],
        out_specs=[_row_spec(128), _row_spec(128)],
        out_shape=[jax.ShapeDtypeStruct((NP2, 128), jnp.float32),
                   jax.ShapeDtypeStruct((NP2, 128), jnp.float32)],
    )(deg2, x2, w1d, rex)


def _tc_combine(acc2, hp, dis, b, wd):
    return pl.pallas_call(
        _combine_body,
        grid=(_GRID,),
        in_specs=[pl.BlockSpec((2, _BLK, 128), lambda i: (0, i, 0)),
                  _row_spec(128), _row_spec(128), _full_spec((1, 128)),
                  _full_spec((128, 128))],
        out_specs=[_row_spec(128)],
        out_shape=[jax.ShapeDtypeStruct((NP2, 128), jnp.float32)],
    )(acc2, hp, dis, b, wd)[0]


def _tc_final(acc2, hp, dis, b):
    return pl.pallas_call(
        _final_body,
        grid=(_GRID,),
        in_specs=[pl.BlockSpec((2, _BLK, 128), lambda i: (0, i, 0)),
                  _row_spec(128), _row_spec(128), _full_spec((1, 128))],
        out_specs=[_row_spec(128)],
        out_shape=[jax.ShapeDtypeStruct((NP2, 128), jnp.float32)],
    )(acc2, hp, dis, b)[0]


def _blockdiag(w):
    fi, fo = w.shape
    z = jnp.zeros((fi, fo), jnp.float32)
    return jnp.concatenate([
        jnp.concatenate([w, z], axis=1),
        jnp.concatenate([z, w], axis=1),
    ], axis=0)


def kernel(x, edge_index, W1, b1, W2, b2, W3, b3):
    ones16 = jnp.ones((K, 16), jnp.float32)
    z16 = jnp.zeros((NP, 16), jnp.float32)
    z64 = jnp.zeros((NP, FP), jnp.float32)
    x2 = jnp.pad(x, ((0, NP - N), (0, 0))).reshape(NP2, 2 * F_IN)
    w3p = jnp.pad(W3, ((0, 0), (0, FP - C)))
    w1d = _blockdiag(W1)
    w2d = _blockdiag(W2)
    w3d = _blockdiag(w3p)
    rex = jnp.zeros((32, 128), jnp.float32)
    rex = rex.at[0, :64].set(1.0).at[16, 64:].set(1.0)
    b1p = jnp.concatenate([b1, b1]).reshape(1, 128)
    b2p = jnp.concatenate([b2, b2]).reshape(1, 128)
    b3f = jnp.pad(b3, (0, FP - C))
    b3p = jnp.concatenate([b3f, b3f]).reshape(1, 128)

    eif = edge_index.reshape(2 * E)
    deg = _sc_degree(eif, ones16, z16)
    deg2 = deg.reshape(NC, NP2, 32)
    dis, hp1 = _tc_prep(deg2, x2, w1d, rex)

    acc1 = _sc_msg(hp1.reshape(NP, FP), eif, z64)
    hp2 = _tc_combine(acc1.reshape(NC, NP2, 128), hp1, dis, b1p, w2d)

    acc2 = _sc_msg(hp2.reshape(NP, FP), eif, z64)
    hp3 = _tc_combine(acc2.reshape(NC, NP2, 128), hp2, dis, b2p, w3d)

    acc3 = _sc_msg(hp3.reshape(NP, FP), eif, z64)
    out = _tc_final(acc3.reshape(NC, NP2, 128), hp3, dis, b3p)
    return out.reshape(NP, FP)[:N, :C]
